# Initial kernel scaffold; baseline (speedup 1.0000x reference)
import jax
import jax.numpy as jnp
from jax.experimental import pallas as pl


def kernel(src, W_e1, W_e2, W_e3, W_e4, W_emb, W_h1a, W_h1b, W_h1s, W_h2a, W_h2b, W_h2s):
    raise NotImplementedError("write your pallas kernel here")


# speedup vs baseline: 18.1385x; 18.1385x over previous
import functools

import jax
import jax.numpy as jnp
from jax import lax
from jax.experimental import pallas as pl
from jax.experimental.pallas import tpu as pltpu
from jax.experimental.pallas import tpu_sc as plsc

_B, _N, _K = 2, 4096, 20
_BN = _B * _N
_NC, _NS = 2, 16
_NW = _NC * _NS
_PPW = _BN // _NW
_BLKR = 256
_BLKP = 512
_LRELU = 0.2


def _lrelu(x):
    return jnp.where(x >= 0, x, _LRELU * x)


def _knn_body(xyzr_ref, xyzc_ref, out_ref):
    b = pl.program_id(0)
    r = xyzr_ref[0]
    c = xyzc_ref[0]
    xxr = jnp.sum(r * r, axis=1, keepdims=True)
    xxc = jnp.sum(c * c, axis=0, keepdims=True)
    dist = xxr - 2.0 * jnp.dot(r, c, preferred_element_type=jnp.float32) + xxc
    coliota = lax.broadcasted_iota(jnp.int32, (_BLKR, _N), 1)
    big = jnp.float32(jnp.inf)
    for t in range(_K):
        m = jnp.min(dist, axis=1, keepdims=True)
        sel = dist == m
        idxt = jnp.min(jnp.where(sel, coliota, _N), axis=1, keepdims=True)
        out_ref[:, t:t + 1] = idxt + b * _N
        dist = jnp.where(coliota == idxt, big, dist)


def _knn(xyz_bn8, xyz_b8n):
    nb = _N // _BLKR
    return pl.pallas_call(
        _knn_body,
        grid=(_B, nb),
        in_specs=[
            pl.BlockSpec((1, _BLKR, 8), lambda b, r: (b, r, 0)),
            pl.BlockSpec((1, 8, _N), lambda b, r: (b, 0, 0)),
        ],
        out_specs=pl.BlockSpec((_BLKR, _K), lambda b, r: (b * nb + r, 0)),
        out_shape=jax.ShapeDtypeStruct((_BN, _K), jnp.int32),
    )(xyz_bn8, xyz_b8n)


def _hog_sc(xyz8_flat, idx32_flat):
    mesh = plsc.VectorSubcoreMesh(core_axis_name="c", subcore_axis_name="s")

    @functools.partial(
        pl.kernel,
        out_type=[jax.ShapeDtypeStruct((_BN * 32,), jnp.float32)] * 3,
        mesh=mesh,
        compiler_params=pltpu.CompilerParams(needs_layout_passes=False),
        scratch_types=[
            pltpu.VMEM((_BN * 8,), jnp.float32),
            pltpu.VMEM((_PPW * 32,), jnp.int32),
            pltpu.VMEM((_PPW * 32,), jnp.float32),
            pltpu.VMEM((_PPW * 32,), jnp.float32),
            pltpu.VMEM((_PPW * 32,), jnp.float32),
        ],
    )
    def k(xyz_hbm, idx_hbm, dx_hbm, dy_hbm, dz_hbm, tab_v, idx_v, dx_v, dy_v, dz_v):
        w = lax.axis_index("s") * _NC + lax.axis_index("c")
        base = w * _PPW
        pltpu.sync_copy(xyz_hbm, tab_v)
        pltpu.sync_copy(idx_hbm.at[pl.ds(base * 32, _PPW * 32)], idx_v)
        outs = (dx_v, dy_v, dz_v)

        def body(p, carry):
            ia = idx_v[pl.ds(p * 32, 16)]
            ib = idx_v[pl.ds(p * 32 + 16, 16)]
            ctr = jnp.full((16,), (base + p) * 8, dtype=jnp.int32)
            for ch in range(3):
                cg = plsc.load_gather(tab_v, [ctr + ch])
                for half, iv in ((0, ia), (1, ib)):
                    g = plsc.load_gather(tab_v, [iv * 8 + ch])
                    outs[ch][pl.ds(p * 32 + half * 16, 16)] = g - cg
            return carry

        lax.fori_loop(0, _PPW, body, 0)
        pltpu.sync_copy(dx_v, dx_hbm.at[pl.ds(base * 32, _PPW * 32)])
        pltpu.sync_copy(dy_v, dy_hbm.at[pl.ds(base * 32, _PPW * 32)])
        pltpu.sync_copy(dz_v, dz_hbm.at[pl.ds(base * 32, _PPW * 32)])

    return k(xyz8_flat, idx32_flat)


def _gathermax(p_tab, idxg, d, pc):
    nchunks = _PPW // pc
    npairs = nchunks // 2
    cl = pc * _K
    mesh = plsc.VectorSubcoreMesh(core_axis_name="c", subcore_axis_name="s")

    @functools.partial(
        pl.kernel,
        out_type=jax.ShapeDtypeStruct((_BN * d,), jnp.float32),
        mesh=mesh,
        compiler_params=pltpu.CompilerParams(
            needs_layout_passes=False, use_tc_tiling_on_sc=False),
        scratch_types=[
            pltpu.VMEM((nchunks, cl), jnp.int32),
            pltpu.VMEM((cl, d), jnp.float32),
            pltpu.VMEM((cl, d), jnp.float32),
            pltpu.VMEM((_PPW * d,), jnp.float32),
            pltpu.SemaphoreType.DMA,
            pltpu.SemaphoreType.DMA,
        ],
    )
    def k(p_hbm, idx_hbm, m_hbm, idx_v, buf_a, buf_b, out_v, sem_a, sem_b):
        w = lax.axis_index("s") * _NC + lax.axis_index("c")
        pltpu.sync_copy(idx_hbm.at[pl.ds(w * nchunks, nchunks)], idx_v)

        def start(c, buf, sem):
            pltpu.make_async_copy(p_hbm.at[idx_v.at[c]], buf, sem).start()

        def wait(c, buf, sem):
            pltpu.make_async_copy(p_hbm.at[idx_v.at[c]], buf, sem).wait()

        def compute(c, buf):
            for p in range(pc):
                def lane(g, carry, p=p):
                    acc = buf[p * _K, pl.ds(g * 16, 16)]
                    for kk in range(1, _K):
                        acc = jnp.maximum(acc, buf[p * _K + kk, pl.ds(g * 16, 16)])
                    out_v[pl.ds((c * pc + p) * d + g * 16, 16)] = acc
                    return carry
                lax.fori_loop(0, d // 16, lane, 0)

        start(0, buf_a, sem_a)

        def pair(j2, carry):
            c0 = j2 * 2
            start(c0 + 1, buf_b, sem_b)
            wait(c0, buf_a, sem_a)
            compute(c0, buf_a)

            @pl.when(j2 + 1 < npairs)
            def _():
                start(c0 + 2, buf_a, sem_a)

            wait(c0 + 1, buf_b, sem_b)
            compute(c0 + 1, buf_b)
            return carry

        lax.fori_loop(0, npairs, pair, 0)
        pltpu.sync_copy(out_v, m_hbm.at[pl.ds(w * _PPW * d, _PPW * d)])

    return k(p_tab, idxg).reshape(_BN, d)


def _feat_body(dx_ref, dy_ref, dz_ref, srcp_ref, ws_ref, wh_ref, a_ref, p_ref):
    dx = dx_ref[...]
    dy = dy_ref[...]
    dz = dz_ref[...]
    lane = lax.broadcasted_iota(jnp.int32, (_BLKP, 32), 1)
    valid = (lane < 16) | (lane >= 28)
    mag = jnp.sqrt(dx * dx + dy * dy + dz * dz + 1e-12)
    phi = jnp.arctan2(dy, dx)
    pi = jnp.float32(3.14159265358979323846)
    bini = jnp.clip(jnp.floor((phi + pi) / (2.0 * pi / 18.0)), 0.0, 17.0).astype(jnp.int32)
    wmag = jnp.where(valid, mag, 0.0)
    bin3 = bini[:, None, :]
    w3 = wmag[:, None, :]
    biota = lax.broadcasted_iota(jnp.int32, (_BLKP, 18, 32), 1)
    hog = jnp.sum(jnp.where(bin3 == biota, w3, 0.0), axis=2)
    hog = hog / (jnp.sum(hog, axis=1, keepdims=True) + 1e-6)
    ap = (jnp.dot(srcp_ref[...], ws_ref[...], preferred_element_type=jnp.float32)
          + jnp.dot(hog, wh_ref[...], preferred_element_type=jnp.float32))
    a_ref[...] = ap[:, :64]
    p_ref[...] = ap[:, 64:]


def _feat(dxh, dyh, dzh, src_pm, w1s, w1h):
    nb = _BN // _BLKP
    return pl.pallas_call(
        _feat_body,
        grid=(nb,),
        in_specs=[
            pl.BlockSpec((_BLKP, 32), lambda i: (i, 0)),
            pl.BlockSpec((_BLKP, 32), lambda i: (i, 0)),
            pl.BlockSpec((_BLKP, 32), lambda i: (i, 0)),
            pl.BlockSpec((_BLKP, 9), lambda i: (i, 0)),
            pl.BlockSpec((9, 128), lambda i: (0, 0)),
            pl.BlockSpec((18, 128), lambda i: (0, 0)),
        ],
        out_specs=[
            pl.BlockSpec((_BLKP, 64), lambda i: (i, 0)),
            pl.BlockSpec((_BLKP, 64), lambda i: (i, 0)),
        ],
        out_shape=[
            jax.ShapeDtypeStruct((_BN, 64), jnp.float32),
            jax.ShapeDtypeStruct((_BN, 64), jnp.float32),
        ],
    )(dxh, dyh, dzh, src_pm, w1s, w1h)


def _layer(a, m, wp, dout):
    din = a.shape[1]

    def body(a_ref, m_ref, w_ref, x_ref, a2_ref, p2_ref):
        x = _lrelu(a_ref[...] + m_ref[...])
        ap = jnp.dot(x, w_ref[...], preferred_element_type=jnp.float32)
        x_ref[...] = x
        a2_ref[...] = ap[:, :dout]
        p2_ref[...] = ap[:, dout:]

    nb = _BN // _BLKP
    return pl.pallas_call(
        body,
        grid=(nb,),
        in_specs=[
            pl.BlockSpec((_BLKP, din), lambda i: (i, 0)),
            pl.BlockSpec((_BLKP, din), lambda i: (i, 0)),
            pl.BlockSpec((din, 2 * dout), lambda i: (0, 0)),
        ],
        out_specs=[
            pl.BlockSpec((_BLKP, din), lambda i: (i, 0)),
            pl.BlockSpec((_BLKP, dout), lambda i: (i, 0)),
            pl.BlockSpec((_BLKP, dout), lambda i: (i, 0)),
        ],
        out_shape=[
            jax.ShapeDtypeStruct((_BN, din), jnp.float32),
            jax.ShapeDtypeStruct((_BN, dout), jnp.float32),
            jax.ShapeDtypeStruct((_BN, dout), jnp.float32),
        ],
    )(a, m, wp)


def _emb_body(a4_ref, m4_ref, x1_ref, x2_ref, x3_ref, we_ref, emb_ref, glob_ref):
    nb = pl.program_id(1)
    x4 = _lrelu(a4_ref[...] + m4_ref[...])
    glob_ref = glob_ref.at[0]
    we = we_ref[...]
    pre = (jnp.dot(x1_ref[...], we[0:64], preferred_element_type=jnp.float32)
           + jnp.dot(x2_ref[...], we[64:128], preferred_element_type=jnp.float32)
           + jnp.dot(x3_ref[...], we[128:256], preferred_element_type=jnp.float32)
           + jnp.dot(x4, we[256:512], preferred_element_type=jnp.float32))
    emb = _lrelu(pre)
    emb_ref[...] = emb
    bmax = jnp.max(emb, axis=0, keepdims=True)

    @pl.when(nb == 0)
    def _():
        glob_ref[...] = bmax

    @pl.when(nb != 0)
    def _():
        glob_ref[...] = jnp.maximum(glob_ref[...], bmax)


def _embed(a4, m4, x1, x2, x3, w_emb):
    nb = _N // _BLKP
    return pl.pallas_call(
        _emb_body,
        grid=(_B, nb),
        in_specs=[
            pl.BlockSpec((_BLKP, 256), lambda b, i: (b * nb + i, 0)),
            pl.BlockSpec((_BLKP, 256), lambda b, i: (b * nb + i, 0)),
            pl.BlockSpec((_BLKP, 64), lambda b, i: (b * nb + i, 0)),
            pl.BlockSpec((_BLKP, 64), lambda b, i: (b * nb + i, 0)),
            pl.BlockSpec((_BLKP, 128), lambda b, i: (b * nb + i, 0)),
            pl.BlockSpec((512, 1024), lambda b, i: (0, 0)),
        ],
        out_specs=[
            pl.BlockSpec((_BLKP, 1024), lambda b, i: (b * nb + i, 0)),
            pl.BlockSpec((1, 1, 1024), lambda b, i: (b, 0, 0)),
        ],
        out_shape=[
            jax.ShapeDtypeStruct((_BN, 1024), jnp.float32),
            jax.ShapeDtypeStruct((_B, 1, 1024), jnp.float32),
        ],
    )(a4, m4, x1, x2, x3, w_emb)


def _head_body(emb_ref, xyz_ref, glob_ref, wae_ref, wax_ref, wag_ref,
               wse_ref, wsx_ref, wsg_ref, whb_ref, w2a_ref, w2b_ref, w2s_ref,
               out_ref):
    e = emb_ref[...]
    xy = xyz_ref[...]
    g = glob_ref[0]
    ga = jnp.dot(g, wag_ref[...], preferred_element_type=jnp.float32)
    gs = jnp.dot(g, wsg_ref[...], preferred_element_type=jnp.float32)
    t = jnp.maximum(
        jnp.dot(e, wae_ref[...], preferred_element_type=jnp.float32)
        + jnp.dot(xy, wax_ref[...], preferred_element_type=jnp.float32) + ga, 0.0)
    s = (jnp.dot(e, wse_ref[...], preferred_element_type=jnp.float32)
         + jnp.dot(xy, wsx_ref[...], preferred_element_type=jnp.float32) + gs)
    o1 = jnp.maximum(jnp.dot(t, whb_ref[...], preferred_element_type=jnp.float32) + s, 0.0)
    t2 = jnp.maximum(jnp.dot(o1, w2a_ref[...], preferred_element_type=jnp.float32), 0.0)
    out_ref[...] = (jnp.dot(t2, w2b_ref[...], preferred_element_type=jnp.float32)
                    + jnp.dot(o1, w2s_ref[...], preferred_element_type=jnp.float32))


def _head(emb, xyz8, glob, wae, wax, wag, wse, wsx, wsg, whb, w2a, w2b, w2s):
    nb = _N // _BLKP
    wspec = lambda r, c: pl.BlockSpec((r, c), lambda b, i: (0, 0))
    return pl.pallas_call(
        _head_body,
        grid=(_B, nb),
        in_specs=[
            pl.BlockSpec((_BLKP, 1024), lambda b, i: (b * nb + i, 0)),
            pl.BlockSpec((_BLKP, 8), lambda b, i: (b * nb + i, 0)),
            pl.BlockSpec((1, 1, 1024), lambda b, i: (b, 0, 0)),
            wspec(1024, 512), wspec(8, 512), wspec(1024, 512),
            wspec(1024, 256), wspec(8, 256), wspec(1024, 256),
            wspec(512, 256), wspec(256, 128), wspec(128, 13), wspec(256, 13),
        ],
        out_specs=pl.BlockSpec((_BLKP, 13), lambda b, i: (b * nb + i, 0)),
        out_shape=jax.ShapeDtypeStruct((_BN, 13), jnp.float32),
    )(emb, xyz8, glob, wae, wax, wag, wse, wsx, wsg, whb, w2a, w2b, w2s)


def _split_ec(w):
    c = w.shape[0] // 2
    return jnp.concatenate([w[:c] - w[c:], w[c:]], axis=1)


def kernel(src, W_e1, W_e2, W_e3, W_e4, W_emb, W_h1a, W_h1b, W_h1s, W_h2a, W_h2b, W_h2s):
    f32 = jnp.float32
    src_pm = src.transpose(0, 2, 1).reshape(_BN, 9)
    xyz8 = jnp.concatenate([src_pm[:, :3], jnp.zeros((_BN, 5), f32)], axis=1)
    xyz_bn8 = xyz8.reshape(_B, _N, 8)
    xyz_b8n = xyz_bn8.transpose(0, 2, 1)

    idx = _knn(xyz_bn8, xyz_b8n)

    idx32 = jnp.concatenate([idx[:, :16], idx[:, 4:]], axis=1).reshape(_BN * 32)
    idx80 = idx.reshape(_BN // 4, 80)
    idx40 = idx.reshape(_BN // 2, 40)

    dxh, dyh, dzh = _hog_sc(xyz8.reshape(_BN * 8), idx32)
    dxh = dxh.reshape(_BN, 32)
    dyh = dyh.reshape(_BN, 32)
    dzh = dzh.reshape(_BN, 32)

    w1p = _split_ec(W_e1)
    w2p = _split_ec(W_e2)
    w3p = _split_ec(W_e3)
    w4p = _split_ec(W_e4)

    a1, p1 = _feat(dxh, dyh, dzh, src_pm, w1p[:9], w1p[9:])
    m1 = _gathermax(p1, idx80, 64, 4)
    x1, a2, p2 = _layer(a1, m1, w2p, 64)
    m2 = _gathermax(p2, idx80, 64, 4)
    x2, a3, p3 = _layer(a2, m2, w3p, 128)
    m3 = _gathermax(p3, idx80, 128, 4)
    x3, a4, p4 = _layer(a3, m3, w4p, 256)
    m4 = _gathermax(p4, idx40, 256, 2)

    emb, glob = _embed(a4, m4, x1, x2, x3, W_emb)

    logits = _head(
        emb, xyz8, glob,
        W_h1a[:1024], jnp.pad(W_h1a[1024:1027], ((0, 5), (0, 0))), W_h1a[1027:],
        W_h1s[:1024], jnp.pad(W_h1s[1024:1027], ((0, 5), (0, 0))), W_h1s[1027:],
        W_h1b, W_h2a, W_h2b, W_h2s,
    )
    return logits.reshape(_B, _N, 13).transpose(0, 2, 1)


# speedup vs baseline: 21.8180x; 1.2029x over previous
import functools

import jax
import jax.numpy as jnp
from jax import lax
from jax.experimental import pallas as pl
from jax.experimental.pallas import tpu as pltpu
from jax.experimental.pallas import tpu_sc as plsc

_B, _N, _K = 2, 4096, 20
_NC, _NS = 2, 16
_NW = _NC * _NS
_PPW = _N // _NW
_BLKR = 256
_BLKP = 512
_LRELU = 0.2


def _lrelu(x):
    return jnp.where(x >= 0, x, _LRELU * x)


def _knn_body(xyzr_ref, xyzc_ref, out_ref):
    r = xyzr_ref[...]
    c = xyzc_ref[...]
    xxr = jnp.sum(r * r, axis=1, keepdims=True)
    xxc = jnp.sum(c * c, axis=0, keepdims=True)
    dist = xxr - 2.0 * jnp.dot(r, c, preferred_element_type=jnp.float32) + xxc
    fiota = lax.broadcasted_iota(jnp.int32, (_BLKR, _N), 1).astype(jnp.float32)
    bigf = jnp.float32(3e38)
    inf = jnp.float32(jnp.inf)
    for t in range(_K):
        m = jnp.min(dist, axis=1, keepdims=True)
        fidx = jnp.where(dist == m, fiota, bigf)
        fmin = jnp.min(fidx, axis=1, keepdims=True)
        out_ref[:, t:t + 1] = fmin.astype(jnp.int32)
        dist = jnp.where(fidx == fmin, inf, dist)


def _knn(xyz_n8, xyz_8n):
    nb = _N // _BLKR
    return pl.pallas_call(
        _knn_body,
        grid=(nb,),
        in_specs=[
            pl.BlockSpec((_BLKR, 8), lambda r: (r, 0)),
            pl.BlockSpec((8, _N), lambda r: (0, 0)),
        ],
        out_specs=pl.BlockSpec((_BLKR, _K), lambda r: (r, 0)),
        out_shape=jax.ShapeDtypeStruct((_N, _K), jnp.int32),
    )(xyz_n8, xyz_8n)


def _hog_sc(xyz8_flat, idx32_flat):
    mesh = plsc.VectorSubcoreMesh(core_axis_name="c", subcore_axis_name="s")

    @functools.partial(
        pl.kernel,
        out_type=[jax.ShapeDtypeStruct((_N * 32,), jnp.float32)] * 3,
        mesh=mesh,
        compiler_params=pltpu.CompilerParams(needs_layout_passes=False),
        scratch_types=[
            pltpu.VMEM((_N * 8,), jnp.float32),
            pltpu.VMEM((_PPW * 32,), jnp.int32),
            pltpu.VMEM((_PPW * 32,), jnp.float32),
            pltpu.VMEM((_PPW * 32,), jnp.float32),
            pltpu.VMEM((_PPW * 32,), jnp.float32),
        ],
    )
    def k(xyz_hbm, idx_hbm, dx_hbm, dy_hbm, dz_hbm, tab_v, idx_v, dx_v, dy_v, dz_v):
        w = lax.axis_index("s") * _NC + lax.axis_index("c")
        base = w * _PPW
        pltpu.sync_copy(xyz_hbm, tab_v)
        pltpu.sync_copy(idx_hbm.at[pl.ds(base * 32, _PPW * 32)], idx_v)
        outs = (dx_v, dy_v, dz_v)

        def body(p, carry):
            ia = idx_v[pl.ds(p * 32, 16)]
            ib = idx_v[pl.ds(p * 32 + 16, 16)]
            ctr = jnp.full((16,), (base + p) * 8, dtype=jnp.int32)
            for ch in range(3):
                cg = plsc.load_gather(tab_v, [ctr + ch])
                for half, iv in ((0, ia), (1, ib)):
                    g = plsc.load_gather(tab_v, [iv * 8 + ch])
                    outs[ch][pl.ds(p * 32 + half * 16, 16)] = g - cg
            return carry

        lax.fori_loop(0, _PPW, body, 0)
        pltpu.sync_copy(dx_v, dx_hbm.at[pl.ds(base * 32, _PPW * 32)])
        pltpu.sync_copy(dy_v, dy_hbm.at[pl.ds(base * 32, _PPW * 32)])
        pltpu.sync_copy(dz_v, dz_hbm.at[pl.ds(base * 32, _PPW * 32)])

    return k(xyz8_flat, idx32_flat)


def _gathermax(p_tab, idxg, d, pc):
    nchunks = _PPW // pc
    npairs = nchunks // 2
    cl = pc * _K
    mesh = plsc.VectorSubcoreMesh(core_axis_name="c", subcore_axis_name="s")

    @functools.partial(
        pl.kernel,
        out_type=jax.ShapeDtypeStruct((_N * d,), jnp.float32),
        mesh=mesh,
        compiler_params=pltpu.CompilerParams(
            needs_layout_passes=False, use_tc_tiling_on_sc=False),
        scratch_types=[
            pltpu.VMEM((nchunks, cl), jnp.int32),
            pltpu.VMEM((cl, d), jnp.float32),
            pltpu.VMEM((cl, d), jnp.float32),
            pltpu.VMEM((_PPW * d,), jnp.float32),
            pltpu.SemaphoreType.DMA,
            pltpu.SemaphoreType.DMA,
        ],
    )
    def k(p_hbm, idx_hbm, m_hbm, idx_v, buf_a, buf_b, out_v, sem_a, sem_b):
        w = lax.axis_index("s") * _NC + lax.axis_index("c")
        pltpu.sync_copy(idx_hbm.at[pl.ds(w * nchunks, nchunks)], idx_v)

        def start(c, buf, sem):
            pltpu.make_async_copy(p_hbm.at[idx_v.at[c]], buf, sem).start()

        def wait(c, buf, sem):
            pltpu.make_async_copy(p_hbm.at[idx_v.at[c]], buf, sem).wait()

        def compute(c, buf):
            for p in range(pc):
                def lane(g, carry, p=p):
                    vals = [buf[p * _K + kk, pl.ds(g * 16, 16)] for kk in range(_K)]
                    while len(vals) > 1:
                        nxt = [jnp.maximum(vals[i], vals[i + 1])
                               for i in range(0, len(vals) - 1, 2)]
                        if len(vals) % 2:
                            nxt.append(vals[-1])
                        vals = nxt
                    out_v[pl.ds((c * pc + p) * d + g * 16, 16)] = vals[0]
                    return carry
                lax.fori_loop(0, d // 16, lane, 0)

        start(0, buf_a, sem_a)

        def pair(j2, carry):
            c0 = j2 * 2
            start(c0 + 1, buf_b, sem_b)
            wait(c0, buf_a, sem_a)
            compute(c0, buf_a)

            @pl.when(j2 + 1 < npairs)
            def _():
                start(c0 + 2, buf_a, sem_a)

            wait(c0 + 1, buf_b, sem_b)
            compute(c0 + 1, buf_b)
            return carry

        lax.fori_loop(0, npairs, pair, 0)
        pltpu.sync_copy(out_v, m_hbm.at[pl.ds(w * _PPW * d, _PPW * d)])

    return k(p_tab, idxg).reshape(_N, d)


def _feat_body(dx_ref, dy_ref, dz_ref, srcp_ref, ws_ref, wh_ref, a_ref, p_ref):
    dx = dx_ref[...]
    dy = dy_ref[...]
    dz = dz_ref[...]
    lane = lax.broadcasted_iota(jnp.int32, (_BLKP, 32), 1)
    valid = (lane < 16) | (lane >= 28)
    mag = jnp.sqrt(dx * dx + dy * dy + dz * dz + 1e-12)
    phi = jnp.arctan2(dy, dx)
    pi = jnp.float32(3.14159265358979323846)
    bini = jnp.clip(jnp.floor((phi + pi) / (2.0 * pi / 18.0)), 0.0, 17.0).astype(jnp.int32)
    wmag = jnp.where(valid, mag, 0.0)
    bin3 = bini[:, None, :]
    w3 = wmag[:, None, :]
    biota = lax.broadcasted_iota(jnp.int32, (_BLKP, 18, 32), 1)
    hog = jnp.sum(jnp.where(bin3 == biota, w3, 0.0), axis=2)
    hog = hog / (jnp.sum(hog, axis=1, keepdims=True) + 1e-6)
    ap = (jnp.dot(srcp_ref[...], ws_ref[...], preferred_element_type=jnp.float32)
          + jnp.dot(hog, wh_ref[...], preferred_element_type=jnp.float32))
    a_ref[...] = ap[:, :64]
    p_ref[...] = ap[:, 64:]


def _feat(dxh, dyh, dzh, src_pm, w1s, w1h):
    nb = _N // _BLKP
    return pl.pallas_call(
        _feat_body,
        grid=(nb,),
        in_specs=[
            pl.BlockSpec((_BLKP, 32), lambda i: (i, 0)),
            pl.BlockSpec((_BLKP, 32), lambda i: (i, 0)),
            pl.BlockSpec((_BLKP, 32), lambda i: (i, 0)),
            pl.BlockSpec((_BLKP, 9), lambda i: (i, 0)),
            pl.BlockSpec((9, 128), lambda i: (0, 0)),
            pl.BlockSpec((18, 128), lambda i: (0, 0)),
        ],
        out_specs=[
            pl.BlockSpec((_BLKP, 64), lambda i: (i, 0)),
            pl.BlockSpec((_BLKP, 64), lambda i: (i, 0)),
        ],
        out_shape=[
            jax.ShapeDtypeStruct((_N, 64), jnp.float32),
            jax.ShapeDtypeStruct((_N, 64), jnp.float32),
        ],
    )(dxh, dyh, dzh, src_pm, w1s, w1h)


def _layer(a, m, wp, dout):
    din = a.shape[1]

    def body(a_ref, m_ref, w_ref, x_ref, a2_ref, p2_ref):
        x = _lrelu(a_ref[...] + m_ref[...])
        ap = jnp.dot(x, w_ref[...], preferred_element_type=jnp.float32)
        x_ref[...] = x
        a2_ref[...] = ap[:, :dout]
        p2_ref[...] = ap[:, dout:]

    nb = _N // _BLKP
    return pl.pallas_call(
        body,
        grid=(nb,),
        in_specs=[
            pl.BlockSpec((_BLKP, din), lambda i: (i, 0)),
            pl.BlockSpec((_BLKP, din), lambda i: (i, 0)),
            pl.BlockSpec((din, 2 * dout), lambda i: (0, 0)),
        ],
        out_specs=[
            pl.BlockSpec((_BLKP, din), lambda i: (i, 0)),
            pl.BlockSpec((_BLKP, dout), lambda i: (i, 0)),
            pl.BlockSpec((_BLKP, dout), lambda i: (i, 0)),
        ],
        out_shape=[
            jax.ShapeDtypeStruct((_N, din), jnp.float32),
            jax.ShapeDtypeStruct((_N, dout), jnp.float32),
            jax.ShapeDtypeStruct((_N, dout), jnp.float32),
        ],
    )(a, m, wp)


def _emb_body(a4_ref, m4_ref, x1_ref, x2_ref, x3_ref, we_ref, emb_ref, glob_ref):
    nb = pl.program_id(0)
    glob_ref = glob_ref.at[0]
    x4 = _lrelu(a4_ref[...] + m4_ref[...])
    we = we_ref[...]
    pre = (jnp.dot(x1_ref[...], we[0:64], preferred_element_type=jnp.float32)
           + jnp.dot(x2_ref[...], we[64:128], preferred_element_type=jnp.float32)
           + jnp.dot(x3_ref[...], we[128:256], preferred_element_type=jnp.float32)
           + jnp.dot(x4, we[256:512], preferred_element_type=jnp.float32))
    emb = _lrelu(pre)
    emb_ref[...] = emb
    bmax = jnp.max(emb, axis=0, keepdims=True)

    @pl.when(nb == 0)
    def _():
        glob_ref[...] = bmax

    @pl.when(nb != 0)
    def _():
        glob_ref[...] = jnp.maximum(glob_ref[...], bmax)


def _embed(a4, m4, x1, x2, x3, w_emb):
    nb = _N // _BLKP
    return pl.pallas_call(
        _emb_body,
        grid=(nb,),
        in_specs=[
            pl.BlockSpec((_BLKP, 256), lambda i: (i, 0)),
            pl.BlockSpec((_BLKP, 256), lambda i: (i, 0)),
            pl.BlockSpec((_BLKP, 64), lambda i: (i, 0)),
            pl.BlockSpec((_BLKP, 64), lambda i: (i, 0)),
            pl.BlockSpec((_BLKP, 128), lambda i: (i, 0)),
            pl.BlockSpec((512, 1024), lambda i: (0, 0)),
        ],
        out_specs=[
            pl.BlockSpec((_BLKP, 1024), lambda i: (i, 0)),
            pl.BlockSpec((1, 1, 1024), lambda i: (0, 0, 0)),
        ],
        out_shape=[
            jax.ShapeDtypeStruct((_N, 1024), jnp.float32),
            jax.ShapeDtypeStruct((1, 1, 1024), jnp.float32),
        ],
    )(a4, m4, x1, x2, x3, w_emb)


def _head_body(emb_ref, xyz_ref, glob_ref, wae_ref, wax_ref, wag_ref,
               wse_ref, wsx_ref, wsg_ref, whb_ref, w2a_ref, w2b_ref, w2s_ref,
               out_ref):
    e = emb_ref[...]
    xy = xyz_ref[...]
    g = glob_ref[0]
    ga = jnp.dot(g, wag_ref[...], preferred_element_type=jnp.float32)
    gs = jnp.dot(g, wsg_ref[...], preferred_element_type=jnp.float32)
    t = jnp.maximum(
        jnp.dot(e, wae_ref[...], preferred_element_type=jnp.float32)
        + jnp.dot(xy, wax_ref[...], preferred_element_type=jnp.float32) + ga, 0.0)
    s = (jnp.dot(e, wse_ref[...], preferred_element_type=jnp.float32)
         + jnp.dot(xy, wsx_ref[...], preferred_element_type=jnp.float32) + gs)
    o1 = jnp.maximum(jnp.dot(t, whb_ref[...], preferred_element_type=jnp.float32) + s, 0.0)
    t2 = jnp.maximum(jnp.dot(o1, w2a_ref[...], preferred_element_type=jnp.float32), 0.0)
    out_ref[...] = (jnp.dot(t2, w2b_ref[...], preferred_element_type=jnp.float32)
                    + jnp.dot(o1, w2s_ref[...], preferred_element_type=jnp.float32))


def _head(emb, xyz8, glob, wae, wax, wag, wse, wsx, wsg, whb, w2a, w2b, w2s):
    nb = _N // _BLKP
    wspec = lambda r, c: pl.BlockSpec((r, c), lambda i: (0, 0))
    return pl.pallas_call(
        _head_body,
        grid=(nb,),
        in_specs=[
            pl.BlockSpec((_BLKP, 1024), lambda i: (i, 0)),
            pl.BlockSpec((_BLKP, 8), lambda i: (i, 0)),
            pl.BlockSpec((1, 1, 1024), lambda i: (0, 0, 0)),
            wspec(1024, 512), wspec(8, 512), wspec(1024, 512),
            wspec(1024, 256), wspec(8, 256), wspec(1024, 256),
            wspec(512, 256), wspec(256, 128), wspec(128, 13), wspec(256, 13),
        ],
        out_specs=pl.BlockSpec((_BLKP, 13), lambda i: (i, 0)),
        out_shape=jax.ShapeDtypeStruct((_N, 13), jnp.float32),
    )(emb, xyz8, glob, wae, wax, wag, wse, wsx, wsg, whb, w2a, w2b, w2s)


def _split_ec(w):
    c = w.shape[0] // 2
    return jnp.concatenate([w[:c] - w[c:], w[c:]], axis=1)


def kernel(src, W_e1, W_e2, W_e3, W_e4, W_emb, W_h1a, W_h1b, W_h1s, W_h2a, W_h2b, W_h2s):
    f32 = jnp.float32
    w1p = _split_ec(W_e1)
    w2p = _split_ec(W_e2)
    w3p = _split_ec(W_e3)
    w4p = _split_ec(W_e4)
    wax = jnp.pad(W_h1a[1024:1027], ((0, 5), (0, 0)))
    wsx = jnp.pad(W_h1s[1024:1027], ((0, 5), (0, 0)))

    src_pm = src.transpose(0, 2, 1)
    logits = []
    for b in range(_B):
        sp = src_pm[b]
        xyz8 = jnp.concatenate([sp[:, :3], jnp.zeros((_N, 5), f32)], axis=1)
        idx = _knn(xyz8, xyz8.T)

        idx32 = jnp.concatenate([idx[:, :16], idx[:, 4:]], axis=1).reshape(_N * 32)
        idx80 = idx.reshape(_N // 4, 80)
        idx40 = idx.reshape(_N // 2, 40)

        dxh, dyh, dzh = _hog_sc(xyz8.reshape(_N * 8), idx32)
        dxh = dxh.reshape(_N, 32)
        dyh = dyh.reshape(_N, 32)
        dzh = dzh.reshape(_N, 32)

        a1, p1 = _feat(dxh, dyh, dzh, sp, w1p[:9], w1p[9:])
        m1 = _gathermax(p1, idx80, 64, 4)
        x1, a2, p2 = _layer(a1, m1, w2p, 64)
        m2 = _gathermax(p2, idx80, 64, 4)
        x2, a3, p3 = _layer(a2, m2, w3p, 128)
        m3 = _gathermax(p3, idx80, 128, 4)
        x3, a4, p4 = _layer(a3, m3, w4p, 256)
        m4 = _gathermax(p4, idx40, 256, 2)

        emb, glob = _embed(a4, m4, x1, x2, x3, W_emb)

        logits.append(_head(
            emb, xyz8, glob,
            W_h1a[:1024], wax, W_h1a[1027:],
            W_h1s[:1024], wsx, W_h1s[1027:],
            W_h1b, W_h2a, W_h2b, W_h2s,
        ))
    return jnp.stack(logits, axis=0).transpose(0, 2, 1)


# speedup vs baseline: 22.4460x; 1.0288x over previous
import functools

import jax
import jax.numpy as jnp
from jax import lax
from jax.experimental import pallas as pl
from jax.experimental.pallas import tpu as pltpu
from jax.experimental.pallas import tpu_sc as plsc

_B, _N, _K = 2, 4096, 20
_NC, _NS = 2, 16
_NW = _NC * _NS
_PPW = _N // _NW
_BLKR = 256
_BLKP = 512
_LRELU = 0.2


def _lrelu(x):
    return jnp.where(x >= 0, x, _LRELU * x)


def _knn_body(xyzr_ref, xyzc_ref, out_ref):
    r = xyzr_ref[...]
    c = xyzc_ref[...]
    xxr = jnp.sum(r * r, axis=1, keepdims=True)
    xxc = jnp.sum(c * c, axis=0, keepdims=True)
    dist = xxr - 2.0 * jnp.dot(r, c, preferred_element_type=jnp.float32) + xxc
    fiota = lax.broadcasted_iota(jnp.int32, (_BLKR, _N), 1).astype(jnp.float32)
    bigf = jnp.float32(3e38)
    inf = jnp.float32(jnp.inf)
    for t in range(_K):
        m = jnp.min(dist, axis=1, keepdims=True)
        sel = dist == m
        fidx = jnp.where(sel, fiota, bigf)
        fmin = jnp.min(fidx, axis=1, keepdims=True)
        out_ref[:, t:t + 1] = fmin.astype(jnp.int32)
        dist = jnp.where(sel, inf, dist)


def _knn(xyz_n8, xyz_8n):
    nb = _N // _BLKR
    return pl.pallas_call(
        _knn_body,
        grid=(nb,),
        in_specs=[
            pl.BlockSpec((_BLKR, 8), lambda r: (r, 0)),
            pl.BlockSpec((8, _N), lambda r: (0, 0)),
        ],
        out_specs=pl.BlockSpec((_BLKR, _K), lambda r: (r, 0)),
        out_shape=jax.ShapeDtypeStruct((_N, _K), jnp.int32),
    )(xyz_n8, xyz_8n)


def _hog_sc(xyz8_flat, idx32_flat):
    mesh = plsc.VectorSubcoreMesh(core_axis_name="c", subcore_axis_name="s")

    @functools.partial(
        pl.kernel,
        out_type=[jax.ShapeDtypeStruct((_N * 32,), jnp.float32)] * 3,
        mesh=mesh,
        compiler_params=pltpu.CompilerParams(needs_layout_passes=False),
        scratch_types=[
            pltpu.VMEM((_N * 8,), jnp.float32),
            pltpu.VMEM((_PPW * 32,), jnp.int32),
            pltpu.VMEM((_PPW * 32,), jnp.float32),
            pltpu.VMEM((_PPW * 32,), jnp.float32),
            pltpu.VMEM((_PPW * 32,), jnp.float32),
        ],
    )
    def k(xyz_hbm, idx_hbm, dx_hbm, dy_hbm, dz_hbm, tab_v, idx_v, dx_v, dy_v, dz_v):
        w = lax.axis_index("s") * _NC + lax.axis_index("c")
        base = w * _PPW
        pltpu.sync_copy(xyz_hbm, tab_v)
        pltpu.sync_copy(idx_hbm.at[pl.ds(base * 32, _PPW * 32)], idx_v)
        outs = (dx_v, dy_v, dz_v)

        def body(p, carry):
            ia = idx_v[pl.ds(p * 32, 16)]
            ib = idx_v[pl.ds(p * 32 + 16, 16)]
            ctr = jnp.full((16,), (base + p) * 8, dtype=jnp.int32)
            for ch in range(3):
                cg = plsc.load_gather(tab_v, [ctr + ch])
                for half, iv in ((0, ia), (1, ib)):
                    g = plsc.load_gather(tab_v, [iv * 8 + ch])
                    outs[ch][pl.ds(p * 32 + half * 16, 16)] = g - cg
            return carry

        lax.fori_loop(0, _PPW, body, 0)
        pltpu.sync_copy(dx_v, dx_hbm.at[pl.ds(base * 32, _PPW * 32)])
        pltpu.sync_copy(dy_v, dy_hbm.at[pl.ds(base * 32, _PPW * 32)])
        pltpu.sync_copy(dz_v, dz_hbm.at[pl.ds(base * 32, _PPW * 32)])

    return k(xyz8_flat, idx32_flat)


def _gathermax(p_tab, idxg, d, pc):
    nchunks = _PPW // pc
    npairs = nchunks // 2
    cl = pc * _K
    mesh = plsc.VectorSubcoreMesh(core_axis_name="c", subcore_axis_name="s")

    @functools.partial(
        pl.kernel,
        out_type=jax.ShapeDtypeStruct((_N * d,), jnp.float32),
        mesh=mesh,
        compiler_params=pltpu.CompilerParams(
            needs_layout_passes=False, use_tc_tiling_on_sc=False),
        scratch_types=[
            pltpu.VMEM((nchunks, cl), jnp.int32),
            pltpu.VMEM((cl, d), jnp.float32),
            pltpu.VMEM((cl, d), jnp.float32),
            pltpu.VMEM((_PPW * d,), jnp.float32),
            pltpu.SemaphoreType.DMA,
            pltpu.SemaphoreType.DMA,
        ],
    )
    def k(p_hbm, idx_hbm, m_hbm, idx_v, buf_a, buf_b, out_v, sem_a, sem_b):
        w = lax.axis_index("s") * _NC + lax.axis_index("c")
        pltpu.sync_copy(idx_hbm.at[pl.ds(w * nchunks, nchunks)], idx_v)

        def start(c, buf, sem):
            pltpu.make_async_copy(p_hbm.at[idx_v.at[c]], buf, sem).start()

        def wait(c, buf, sem):
            pltpu.make_async_copy(p_hbm.at[idx_v.at[c]], buf, sem).wait()

        def compute(c, buf):
            for p in range(pc):
                def lane(g, carry, p=p):
                    vals = [buf[p * _K + kk, pl.ds(g * 16, 16)] for kk in range(_K)]
                    while len(vals) > 1:
                        nxt = [jnp.maximum(vals[i], vals[i + 1])
                               for i in range(0, len(vals) - 1, 2)]
                        if len(vals) % 2:
                            nxt.append(vals[-1])
                        vals = nxt
                    out_v[pl.ds((c * pc + p) * d + g * 16, 16)] = vals[0]
                    return carry
                lax.fori_loop(0, d // 16, lane, 0)

        start(0, buf_a, sem_a)

        def pair(j2, carry):
            c0 = j2 * 2
            start(c0 + 1, buf_b, sem_b)
            wait(c0, buf_a, sem_a)
            compute(c0, buf_a)

            @pl.when(j2 + 1 < npairs)
            def _():
                start(c0 + 2, buf_a, sem_a)

            wait(c0 + 1, buf_b, sem_b)
            compute(c0 + 1, buf_b)
            return carry

        lax.fori_loop(0, npairs, pair, 0)
        pltpu.sync_copy(out_v, m_hbm.at[pl.ds(w * _PPW * d, _PPW * d)])

    return k(p_tab, idxg).reshape(_N, d)


def _feat_body(dx_ref, dy_ref, dz_ref, srcp_ref, ws_ref, wh_ref, a_ref, p_ref):
    dx = dx_ref[...]
    dy = dy_ref[...]
    dz = dz_ref[...]
    lane = lax.broadcasted_iota(jnp.int32, (_BLKP, 32), 1)
    valid = (lane < 16) | (lane >= 28)
    mag = jnp.sqrt(dx * dx + dy * dy + dz * dz + 1e-12)
    phi = jnp.arctan2(dy, dx)
    pi = jnp.float32(3.14159265358979323846)
    bini = jnp.clip(jnp.floor((phi + pi) / (2.0 * pi / 18.0)), 0.0, 17.0).astype(jnp.int32)
    wmag = jnp.where(valid, mag, 0.0)
    bin3 = bini[:, None, :]
    w3 = wmag[:, None, :]
    biota = lax.broadcasted_iota(jnp.int32, (_BLKP, 18, 32), 1)
    hog = jnp.sum(jnp.where(bin3 == biota, w3, 0.0), axis=2)
    hog = hog / (jnp.sum(hog, axis=1, keepdims=True) + 1e-6)
    ap = (jnp.dot(srcp_ref[...], ws_ref[...], preferred_element_type=jnp.float32)
          + jnp.dot(hog, wh_ref[...], preferred_element_type=jnp.float32))
    a_ref[...] = ap[:, :64]
    p_ref[...] = ap[:, 64:]


def _feat(dxh, dyh, dzh, src_pm, w1s, w1h):
    nb = _N // _BLKP
    return pl.pallas_call(
        _feat_body,
        grid=(nb,),
        in_specs=[
            pl.BlockSpec((_BLKP, 32), lambda i: (i, 0)),
            pl.BlockSpec((_BLKP, 32), lambda i: (i, 0)),
            pl.BlockSpec((_BLKP, 32), lambda i: (i, 0)),
            pl.BlockSpec((_BLKP, 9), lambda i: (i, 0)),
            pl.BlockSpec((9, 128), lambda i: (0, 0)),
            pl.BlockSpec((18, 128), lambda i: (0, 0)),
        ],
        out_specs=[
            pl.BlockSpec((_BLKP, 64), lambda i: (i, 0)),
            pl.BlockSpec((_BLKP, 64), lambda i: (i, 0)),
        ],
        out_shape=[
            jax.ShapeDtypeStruct((_N, 64), jnp.float32),
            jax.ShapeDtypeStruct((_N, 64), jnp.float32),
        ],
    )(dxh, dyh, dzh, src_pm, w1s, w1h)


def _layer(a, m, wp, dout):
    din = a.shape[1]

    def body(a_ref, m_ref, w_ref, x_ref, a2_ref, p2_ref):
        x = _lrelu(a_ref[...] + m_ref[...])
        ap = jnp.dot(x, w_ref[...], preferred_element_type=jnp.float32)
        x_ref[...] = x
        a2_ref[...] = ap[:, :dout]
        p2_ref[...] = ap[:, dout:]

    nb = _N // _BLKP
    return pl.pallas_call(
        body,
        grid=(nb,),
        in_specs=[
            pl.BlockSpec((_BLKP, din), lambda i: (i, 0)),
            pl.BlockSpec((_BLKP, din), lambda i: (i, 0)),
            pl.BlockSpec((din, 2 * dout), lambda i: (0, 0)),
        ],
        out_specs=[
            pl.BlockSpec((_BLKP, din), lambda i: (i, 0)),
            pl.BlockSpec((_BLKP, dout), lambda i: (i, 0)),
            pl.BlockSpec((_BLKP, dout), lambda i: (i, 0)),
        ],
        out_shape=[
            jax.ShapeDtypeStruct((_N, din), jnp.float32),
            jax.ShapeDtypeStruct((_N, dout), jnp.float32),
            jax.ShapeDtypeStruct((_N, dout), jnp.float32),
        ],
    )(a, m, wp)


def _emb_body(a4_ref, m4_ref, x1_ref, x2_ref, x3_ref, we_ref, emb_ref, glob_ref):
    nb = pl.program_id(0)
    glob_ref = glob_ref.at[0]
    x4 = _lrelu(a4_ref[...] + m4_ref[...])
    we = we_ref[...]
    pre = (jnp.dot(x1_ref[...], we[0:64], preferred_element_type=jnp.float32)
           + jnp.dot(x2_ref[...], we[64:128], preferred_element_type=jnp.float32)
           + jnp.dot(x3_ref[...], we[128:256], preferred_element_type=jnp.float32)
           + jnp.dot(x4, we[256:512], preferred_element_type=jnp.float32))
    emb = _lrelu(pre)
    emb_ref[...] = emb
    bmax = jnp.max(emb, axis=0, keepdims=True)

    @pl.when(nb == 0)
    def _():
        glob_ref[...] = bmax

    @pl.when(nb != 0)
    def _():
        glob_ref[...] = jnp.maximum(glob_ref[...], bmax)


def _embed(a4, m4, x1, x2, x3, w_emb):
    nb = _N // _BLKP
    return pl.pallas_call(
        _emb_body,
        grid=(nb,),
        in_specs=[
            pl.BlockSpec((_BLKP, 256), lambda i: (i, 0)),
            pl.BlockSpec((_BLKP, 256), lambda i: (i, 0)),
            pl.BlockSpec((_BLKP, 64), lambda i: (i, 0)),
            pl.BlockSpec((_BLKP, 64), lambda i: (i, 0)),
            pl.BlockSpec((_BLKP, 128), lambda i: (i, 0)),
            pl.BlockSpec((512, 1024), lambda i: (0, 0)),
        ],
        out_specs=[
            pl.BlockSpec((_BLKP, 1024), lambda i: (i, 0)),
            pl.BlockSpec((1, 1, 1024), lambda i: (0, 0, 0)),
        ],
        out_shape=[
            jax.ShapeDtypeStruct((_N, 1024), jnp.float32),
            jax.ShapeDtypeStruct((1, 1, 1024), jnp.float32),
        ],
    )(a4, m4, x1, x2, x3, w_emb)


def _head_body(emb_ref, xyz_ref, glob_ref, wae_ref, wax_ref, wag_ref,
               wse_ref, wsx_ref, wsg_ref, whb_ref, w2a_ref, w2b_ref, w2s_ref,
               out_ref):
    e = emb_ref[...]
    xy = xyz_ref[...]
    g = glob_ref[0]
    ga = jnp.dot(g, wag_ref[...], preferred_element_type=jnp.float32)
    gs = jnp.dot(g, wsg_ref[...], preferred_element_type=jnp.float32)
    t = jnp.maximum(
        jnp.dot(e, wae_ref[...], preferred_element_type=jnp.float32)
        + jnp.dot(xy, wax_ref[...], preferred_element_type=jnp.float32) + ga, 0.0)
    s = (jnp.dot(e, wse_ref[...], preferred_element_type=jnp.float32)
         + jnp.dot(xy, wsx_ref[...], preferred_element_type=jnp.float32) + gs)
    o1 = jnp.maximum(jnp.dot(t, whb_ref[...], preferred_element_type=jnp.float32) + s, 0.0)
    t2 = jnp.maximum(jnp.dot(o1, w2a_ref[...], preferred_element_type=jnp.float32), 0.0)
    out_ref[...] = (jnp.dot(t2, w2b_ref[...], preferred_element_type=jnp.float32)
                    + jnp.dot(o1, w2s_ref[...], preferred_element_type=jnp.float32))


def _head(emb, xyz8, glob, wae, wax, wag, wse, wsx, wsg, whb, w2a, w2b, w2s):
    nb = _N // _BLKP
    wspec = lambda r, c: pl.BlockSpec((r, c), lambda i: (0, 0))
    return pl.pallas_call(
        _head_body,
        grid=(nb,),
        in_specs=[
            pl.BlockSpec((_BLKP, 1024), lambda i: (i, 0)),
            pl.BlockSpec((_BLKP, 8), lambda i: (i, 0)),
            pl.BlockSpec((1, 1, 1024), lambda i: (0, 0, 0)),
            wspec(1024, 512), wspec(8, 512), wspec(1024, 512),
            wspec(1024, 256), wspec(8, 256), wspec(1024, 256),
            wspec(512, 256), wspec(256, 128), wspec(128, 13), wspec(256, 13),
        ],
        out_specs=pl.BlockSpec((_BLKP, 13), lambda i: (i, 0)),
        out_shape=jax.ShapeDtypeStruct((_N, 13), jnp.float32),
    )(emb, xyz8, glob, wae, wax, wag, wse, wsx, wsg, whb, w2a, w2b, w2s)


def _split_ec(w):
    c = w.shape[0] // 2
    return jnp.concatenate([w[:c] - w[c:], w[c:]], axis=1)


def kernel(src, W_e1, W_e2, W_e3, W_e4, W_emb, W_h1a, W_h1b, W_h1s, W_h2a, W_h2b, W_h2s):
    f32 = jnp.float32
    w1p = _split_ec(W_e1)
    w2p = _split_ec(W_e2)
    w3p = _split_ec(W_e3)
    w4p = _split_ec(W_e4)
    wax = jnp.pad(W_h1a[1024:1027], ((0, 5), (0, 0)))
    wsx = jnp.pad(W_h1s[1024:1027], ((0, 5), (0, 0)))

    src_pm = src.transpose(0, 2, 1)

    st = [{} for _ in range(_B)]
    for b in range(_B):
        s = st[b]
        s["sp"] = src_pm[b]
        s["xyz8"] = jnp.concatenate(
            [s["sp"][:, :3], jnp.zeros((_N, 5), f32)], axis=1)
        idx = _knn(s["xyz8"], s["xyz8"].T)
        s["idx32"] = jnp.concatenate(
            [idx[:, :16], idx[:, 4:]], axis=1).reshape(_N * 32)
        s["idx80"] = idx.reshape(_N // 4, 80)
        s["idx40"] = idx.reshape(_N // 2, 40)
        s["planes"] = _hog_sc(s["xyz8"].reshape(_N * 8), s["idx32"])

    for b in range(_B):
        s = st[b]
        dxh, dyh, dzh = (p.reshape(_N, 32) for p in s["planes"])
        s["a1"], p1 = _feat(dxh, dyh, dzh, s["sp"], w1p[:9], w1p[9:])
        s["m1"] = _gathermax(p1, s["idx80"], 64, 4)
    for b in range(_B):
        s = st[b]
        s["x1"], s["a2"], p2 = _layer(s["a1"], s["m1"], w2p, 64)
        s["m2"] = _gathermax(p2, s["idx80"], 64, 4)
    for b in range(_B):
        s = st[b]
        s["x2"], s["a3"], p3 = _layer(s["a2"], s["m2"], w3p, 128)
        s["m3"] = _gathermax(p3, s["idx80"], 128, 4)
    for b in range(_B):
        s = st[b]
        s["x3"], s["a4"], p4 = _layer(s["a3"], s["m3"], w4p, 256)
        s["m4"] = _gathermax(p4, s["idx40"], 256, 2)
    for b in range(_B):
        s = st[b]
        s["emb"], s["glob"] = _embed(
            s["a4"], s["m4"], s["x1"], s["x2"], s["x3"], W_emb)
    logits = []
    for b in range(_B):
        s = st[b]
        logits.append(_head(
            s["emb"], s["xyz8"], s["glob"],
            W_h1a[:1024], wax, W_h1a[1027:],
            W_h1s[:1024], wsx, W_h1s[1027:],
            W_h1b, W_h2a, W_h2b, W_h2s,
        ))
    return jnp.stack(logits, axis=0).transpose(0, 2, 1)
